# Initial kernel scaffold; baseline (speedup 1.0000x reference)
import jax
import jax.numpy as jnp
from jax.experimental import pallas as pl


def kernel(x, W_router, w_fc, w_gate, w_proj):
    raise NotImplementedError("write your pallas kernel here")


# speedup vs baseline: 1.1383x; 1.1383x over previous
import functools

import jax
import jax.numpy as jnp
from jax import lax
from jax.experimental import pallas as pl
from jax.experimental.pallas import tpu as pltpu
from jax.experimental.pallas import tpu_sc as plsc

B, T, C, E, H = 2, 2048, 1024, 8, 2048
TOPK = 2
CAP = int(1.25 * TOPK * max(1, T / E))
RPE = B * CAP
NSLOT = E * RPE
NTOK = B * T
NASSIGN = TOPK * NTOK
SENT = NTOK
LANES = 128

NC, NS = 2, 16
NW = NC * NS
INV_LEN = 10496
INV_PER_S = INV_LEN // NS
TRASH = NSLOT
A_PER_S = NASSIGN // NS
A_CHUNK = 64
A_NCH = A_PER_S // A_CHUNK
SLOTS_PER_W = NSLOT // NW
D_CHUNK = 64
D_NCH = SLOTS_PER_W // D_CHUNK
TOK_PER_W = NTOK // NW
C_CHUNK = 32
C_NCH = TOK_PER_W // C_CHUNK


def _router_body(x_ref, wp_ref, tri_ref,
                 p0_ref, p1_ref, dst0_ref, dst1_ref, g0_ref, g1_ref):
    b = pl.program_id(0)
    xb = x_ref[0]
    logits = jnp.dot(xb, wp_ref[...], preferred_element_type=jnp.float32)
    col = lax.broadcasted_iota(jnp.int32, (T, LANES), 1)
    valid = col < E
    lg = jnp.where(valid, logits, jnp.float32(-1e30))
    m = jnp.max(lg, axis=1, keepdims=True)
    ex = jnp.where(valid, jnp.exp(lg - m), 0.0)
    s = jnp.sum(ex, axis=1, keepdims=True)
    prob = ex / s
    pn = jnp.where(valid, prob, -1.0)
    v1 = jnp.max(pn, axis=1, keepdims=True)
    e0 = jnp.min(jnp.where(valid & (pn == v1), col, 999), axis=1,
                 keepdims=True)
    oh0 = col == e0
    pn2 = jnp.where(oh0, -1.0, pn)
    v2 = jnp.max(pn2, axis=1, keepdims=True)
    e1 = jnp.min(jnp.where(valid & (pn2 == v2), col, 999), axis=1,
                 keepdims=True)
    oh1 = col == e1
    oh0f = oh0.astype(jnp.float32)
    oh1f = oh1.astype(jnp.float32)
    c0 = jnp.dot(tri_ref[...], oh0f, preferred_element_type=jnp.float32)
    c1 = jnp.dot(tri_ref[...], oh1f, preferred_element_type=jnp.float32)
    tot0 = jnp.sum(oh0f, axis=0, keepdims=True)
    pos0 = jnp.sum(c0 * oh0f, axis=1).astype(jnp.int32)
    pos1 = jnp.sum((c1 + tot0) * oh1f, axis=1).astype(jnp.int32)
    e0s = jnp.sum(jnp.where(oh0, col, 0), axis=1)
    e1s = jnp.sum(jnp.where(oh1, col, 0), axis=1)
    base0 = e0s * RPE + b * CAP
    base1 = e1s * RPE + b * CAP
    p0_ref[0, 0, :] = jnp.sum(jnp.where(oh0, prob, 0.0), axis=1)
    p1_ref[0, 0, :] = jnp.sum(jnp.where(oh1, prob, 0.0), axis=1)
    dst0_ref[0, 0, :] = jnp.where(pos0 < CAP, base0 + pos0, TRASH)
    dst1_ref[0, 0, :] = jnp.where(pos1 < CAP, base1 + pos1, TRASH)
    g0_ref[0, 0, :] = base0 + jnp.minimum(pos0, CAP - 1)
    g1_ref[0, 0, :] = base1 + jnp.minimum(pos1, CAP - 1)


def _router_call(x, wp, tri, interpret=False):
    i32 = jnp.int32
    out_shape = [
        jax.ShapeDtypeStruct((B, 1, T), jnp.float32),
        jax.ShapeDtypeStruct((B, 1, T), jnp.float32),
        jax.ShapeDtypeStruct((B, 1, T), i32),
        jax.ShapeDtypeStruct((B, 1, T), i32),
        jax.ShapeDtypeStruct((B, 1, T), i32),
        jax.ShapeDtypeStruct((B, 1, T), i32),
    ]
    ospec = pl.BlockSpec((1, 1, T), lambda b: (b, 0, 0))
    return pl.pallas_call(
        _router_body,
        grid=(B,),
        in_specs=[
            pl.BlockSpec((1, T, C), lambda b: (b, 0, 0)),
            pl.BlockSpec((C, LANES), lambda b: (0, 0)),
            pl.BlockSpec((T, T), lambda b: (0, 0)),
        ],
        out_specs=[ospec] * 6,
        out_shape=out_shape,
        interpret=interpret,
    )(x, wp, tri)


def _dispatch_body(xz, dstv, srcv, out,
                   inv_sp, fill_v, dst_v, src_v, invc_v, rows_v, sem):
    cid = lax.axis_index("c")
    sid = lax.axis_index("s")
    wid = sid * NC + cid
    splat = jnp.full((16,), SENT, jnp.int32)
    for i in range(INV_PER_S // 16):
        fill_v[pl.ds(i * 16, 16)] = splat
    pltpu.sync_copy(fill_v, inv_sp.at[pl.ds(sid * INV_PER_S, INV_PER_S)])
    plsc.subcore_barrier()
    pltpu.sync_copy(dstv.at[pl.ds(sid * A_NCH, A_NCH)], dst_v)
    pltpu.sync_copy(srcv.at[pl.ds(sid * A_NCH, A_NCH)], src_v)
    for j in range(A_NCH):
        pltpu.sync_copy(src_v.at[j], inv_sp.at[dst_v.at[j]])
    plsc.subcore_barrier()
    base = wid * SLOTS_PER_W
    pltpu.sync_copy(inv_sp.at[pl.ds(base, SLOTS_PER_W)], invc_v)
    for cb in range(D_NCH):
        pltpu.async_copy(
            xz.at[invc_v.at[pl.ds(cb * D_CHUNK, D_CHUNK)]], rows_v, sem
        ).wait()
        pltpu.sync_copy(rows_v, out.at[pl.ds(base + cb * D_CHUNK, D_CHUNK)])


def _dispatch_call(xz, dstv, srcv):
    mesh = plsc.VectorSubcoreMesh(core_axis_name="c", subcore_axis_name="s")
    return pl.kernel(
        _dispatch_body,
        out_type=jax.ShapeDtypeStruct((NSLOT, C), jnp.float32),
        mesh=mesh,
        scratch_types=[
            pltpu.VMEM_SHARED((INV_LEN,), jnp.int32),
            pltpu.VMEM((INV_PER_S,), jnp.int32),
            pltpu.VMEM((A_NCH, A_CHUNK), jnp.int32),
            pltpu.VMEM((A_NCH, A_CHUNK), jnp.int32),
            pltpu.VMEM((SLOTS_PER_W,), jnp.int32),
            pltpu.VMEM((D_CHUNK, C), jnp.float32),
            pltpu.SemaphoreType.DMA,
        ],
    )(xz, dstv, srcv)


RBLK = 256
NRB = RPE // RBLK


def _glu_body(in_ref, wf_ref, wg_ref, wp_ref, out_ref):
    a = in_ref[0]
    g = jnp.dot(a, wg_ref[0], preferred_element_type=jnp.float32)
    h = jnp.dot(a, wf_ref[0], preferred_element_type=jnp.float32)
    p = jax.nn.silu(g) * h
    out_ref[0] = jnp.dot(p, wp_ref[0], preferred_element_type=jnp.float32)


def _glu_call(ein, w_fc, w_gate, w_proj, interpret=False):
    return pl.pallas_call(
        _glu_body,
        grid=(E, NRB),
        in_specs=[
            pl.BlockSpec((1, RBLK, C), lambda e, r: (e, r, 0)),
            pl.BlockSpec((1, C, H), lambda e, r: (e, 0, 0)),
            pl.BlockSpec((1, C, H), lambda e, r: (e, 0, 0)),
            pl.BlockSpec((1, H, C), lambda e, r: (e, 0, 0)),
        ],
        out_specs=pl.BlockSpec((1, RBLK, C), lambda e, r: (e, r, 0)),
        out_shape=jax.ShapeDtypeStruct((E, RPE, C), jnp.float32),
        interpret=interpret,
    )(ein, w_fc, w_gate, w_proj)


def _combine_body(eo, gi0, gi1, pr0, pr1, y,
                  gi0_v, gi1_v, p0_v, p1_v, g0_v, g1_v, o_v, sem):
    cid = lax.axis_index("c")
    sid = lax.axis_index("s")
    wid = sid * NC + cid
    base = wid * TOK_PER_W
    pltpu.sync_copy(gi0.at[pl.ds(base, TOK_PER_W)], gi0_v)
    pltpu.sync_copy(gi1.at[pl.ds(base, TOK_PER_W)], gi1_v)
    pltpu.sync_copy(pr0.at[pl.ds(base * 16, TOK_PER_W * 16)], p0_v)
    pltpu.sync_copy(pr1.at[pl.ds(base * 16, TOK_PER_W * 16)], p1_v)
    for cb in range(C_NCH):
        cpa = pltpu.async_copy(
            eo.at[gi0_v.at[pl.ds(cb * C_CHUNK, C_CHUNK)]], g0_v, sem)
        cpb = pltpu.async_copy(
            eo.at[gi1_v.at[pl.ds(cb * C_CHUNK, C_CHUNK)]], g1_v, sem)
        cpa.wait()
        cpb.wait()

        def tok_body(j, carry, cb=cb):
            psl = pl.ds((cb * C_CHUNK + j) * 16, 16)
            sp0 = p0_v[psl]
            sp1 = p1_v[psl]
            for ch in range(C // 16):
                sl = pl.ds(ch * 16, 16)
                o_v[j, sl] = sp0 * g0_v[j, sl] + sp1 * g1_v[j, sl]
            return carry

        lax.fori_loop(0, C_CHUNK, tok_body, 0)
        pltpu.sync_copy(o_v, y.at[pl.ds(base + cb * C_CHUNK, C_CHUNK)])


def _combine_call(eo, gi0, gi1, pr0, pr1):
    mesh = plsc.VectorSubcoreMesh(core_axis_name="c", subcore_axis_name="s")
    return pl.kernel(
        _combine_body,
        out_type=jax.ShapeDtypeStruct((NTOK, C), jnp.float32),
        mesh=mesh,
        scratch_types=[
            pltpu.VMEM((TOK_PER_W,), jnp.int32),
            pltpu.VMEM((TOK_PER_W,), jnp.int32),
            pltpu.VMEM((TOK_PER_W * 16,), jnp.float32),
            pltpu.VMEM((TOK_PER_W * 16,), jnp.float32),
            pltpu.VMEM((C_CHUNK, C), jnp.float32),
            pltpu.VMEM((C_CHUNK, C), jnp.float32),
            pltpu.VMEM((C_CHUNK, C), jnp.float32),
            pltpu.SemaphoreType.DMA,
        ],
    )(eo, gi0, gi1, pr0, pr1)


def kernel(x, W_router, w_fc, w_gate, w_proj):
    wp = jnp.pad(W_router, ((0, 0), (0, LANES - E)))
    tri = jnp.tril(jnp.ones((T, T), jnp.float32), -1)
    p0, p1, dst0, dst1, g0, g1 = _router_call(x, wp, tri)
    xz = jnp.concatenate(
        [x.reshape(NTOK, C), jnp.zeros((8, C), x.dtype)], axis=0)
    dstv = jnp.concatenate(
        [dst0.reshape(-1), dst1.reshape(-1)]).reshape(NS * A_NCH, A_CHUNK)
    srcv = jnp.tile(jnp.arange(NTOK, dtype=jnp.int32),
                    TOPK).reshape(NS * A_NCH, A_CHUNK)
    ein = _dispatch_call(xz, dstv, srcv)
    eo = _glu_call(ein.reshape(E, RPE, C), w_fc, w_gate, w_proj)
    p0x = jnp.broadcast_to(p0.reshape(NTOK, 1), (NTOK, 16)).reshape(NTOK * 16)
    p1x = jnp.broadcast_to(p1.reshape(NTOK, 1), (NTOK, 16)).reshape(NTOK * 16)
    y = _combine_call(eo.reshape(NSLOT, C), g0.reshape(-1), g1.reshape(-1),
                      p0x, p1x)
    return y.reshape(B, T, C)


# speedup vs baseline: 1.1495x; 1.0098x over previous
import functools

import jax
import jax.numpy as jnp
from jax import lax
from jax.experimental import pallas as pl
from jax.experimental.pallas import tpu as pltpu
from jax.experimental.pallas import tpu_sc as plsc

B, T, C, E, H = 2, 2048, 1024, 8, 2048
TOPK = 2
CAP = int(1.25 * TOPK * max(1, T / E))
RPE = B * CAP
NSLOT = E * RPE
NTOK = B * T
NASSIGN = TOPK * NTOK
SENT = NTOK
LANES = 128

NC, NS = 2, 16
NW = NC * NS
INV_LEN = 10496
INV_PER_S = INV_LEN // NS
TRASH = NSLOT
A_PER_S = NASSIGN // NS
A_CHUNK = 64
A_NCH = A_PER_S // A_CHUNK
SLOTS_PER_W = NSLOT // NW
D_CHUNK = 40
D_NCH = SLOTS_PER_W // D_CHUNK
TOK_PER_W = NTOK // NW
C_CHUNK = 16
C_NCH = TOK_PER_W // C_CHUNK


def _router_body(x_ref, wp_ref, tri_ref,
                 p0_ref, p1_ref, dst0_ref, dst1_ref, g0_ref, g1_ref):
    b = pl.program_id(0)
    xb = x_ref[0]
    logits = jnp.dot(xb, wp_ref[...], preferred_element_type=jnp.float32)
    col = lax.broadcasted_iota(jnp.int32, (T, LANES), 1)
    valid = col < E
    lg = jnp.where(valid, logits, jnp.float32(-1e30))
    m = jnp.max(lg, axis=1, keepdims=True)
    ex = jnp.where(valid, jnp.exp(lg - m), 0.0)
    s = jnp.sum(ex, axis=1, keepdims=True)
    prob = ex / s
    pn = jnp.where(valid, prob, -1.0)
    v1 = jnp.max(pn, axis=1, keepdims=True)
    e0 = jnp.min(jnp.where(valid & (pn == v1), col, 999), axis=1,
                 keepdims=True)
    oh0 = col == e0
    pn2 = jnp.where(oh0, -1.0, pn)
    v2 = jnp.max(pn2, axis=1, keepdims=True)
    e1 = jnp.min(jnp.where(valid & (pn2 == v2), col, 999), axis=1,
                 keepdims=True)
    oh1 = col == e1
    oh0f = oh0.astype(jnp.float32)
    oh1f = oh1.astype(jnp.float32)
    c0 = jnp.dot(tri_ref[...], oh0f, preferred_element_type=jnp.float32)
    c1 = jnp.dot(tri_ref[...], oh1f, preferred_element_type=jnp.float32)
    tot0 = jnp.sum(oh0f, axis=0, keepdims=True)
    pos0 = jnp.sum(c0 * oh0f, axis=1).astype(jnp.int32)
    pos1 = jnp.sum((c1 + tot0) * oh1f, axis=1).astype(jnp.int32)
    e0s = jnp.sum(jnp.where(oh0, col, 0), axis=1)
    e1s = jnp.sum(jnp.where(oh1, col, 0), axis=1)
    base0 = e0s * RPE + b * CAP
    base1 = e1s * RPE + b * CAP
    p0_ref[0, 0, :] = jnp.sum(jnp.where(oh0, prob, 0.0), axis=1)
    p1_ref[0, 0, :] = jnp.sum(jnp.where(oh1, prob, 0.0), axis=1)
    dst0_ref[0, 0, :] = jnp.where(pos0 < CAP, base0 + pos0, TRASH)
    dst1_ref[0, 0, :] = jnp.where(pos1 < CAP, base1 + pos1, TRASH)
    g0_ref[0, 0, :] = base0 + jnp.minimum(pos0, CAP - 1)
    g1_ref[0, 0, :] = base1 + jnp.minimum(pos1, CAP - 1)


def _router_call(x, wp, tri, interpret=False):
    i32 = jnp.int32
    out_shape = [
        jax.ShapeDtypeStruct((B, 1, T), jnp.float32),
        jax.ShapeDtypeStruct((B, 1, T), jnp.float32),
        jax.ShapeDtypeStruct((B, 1, T), i32),
        jax.ShapeDtypeStruct((B, 1, T), i32),
        jax.ShapeDtypeStruct((B, 1, T), i32),
        jax.ShapeDtypeStruct((B, 1, T), i32),
    ]
    ospec = pl.BlockSpec((1, 1, T), lambda b: (b, 0, 0))
    return pl.pallas_call(
        _router_body,
        grid=(B,),
        in_specs=[
            pl.BlockSpec((1, T, C), lambda b: (b, 0, 0)),
            pl.BlockSpec((C, LANES), lambda b: (0, 0)),
            pl.BlockSpec((T, T), lambda b: (0, 0)),
        ],
        out_specs=[ospec] * 6,
        out_shape=out_shape,
        interpret=interpret,
    )(x, wp, tri)


def _dispatch_body(xz, dstv, srcv, out,
                   inv_sp, fill_v, dst_v, src_v, invc_v, rows_v,
                   gs0, gs1, ss0, ss1):
    cid = lax.axis_index("c")
    sid = lax.axis_index("s")
    wid = sid * NC + cid
    splat = jnp.full((16,), SENT, jnp.int32)
    for i in range(INV_PER_S // 16):
        fill_v[pl.ds(i * 16, 16)] = splat
    pltpu.sync_copy(fill_v, inv_sp.at[pl.ds(sid * INV_PER_S, INV_PER_S)])
    plsc.subcore_barrier()
    pltpu.sync_copy(dstv.at[pl.ds(sid * A_NCH, A_NCH)], dst_v)
    pltpu.sync_copy(srcv.at[pl.ds(sid * A_NCH, A_NCH)], src_v)
    for j in range(A_NCH):
        pltpu.sync_copy(src_v.at[j], inv_sp.at[dst_v.at[j]])
    plsc.subcore_barrier()
    base = wid * SLOTS_PER_W
    pltpu.sync_copy(inv_sp.at[pl.ds(base, SLOTS_PER_W)], invc_v)
    gsem = (gs0, gs1)
    ssem = (ss0, ss1)

    def gather(cb, b):
        return pltpu.async_copy(
            xz.at[invc_v.at[pl.ds(cb * D_CHUNK, D_CHUNK)]],
            rows_v.at[b], gsem[b])

    def store(cb, b):
        return pltpu.async_copy(
            rows_v.at[b], out.at[pl.ds(base + cb * D_CHUNK, D_CHUNK)],
            ssem[b])

    g = [None, None]
    s = [None, None]
    g[0] = gather(0, 0)
    g[1] = gather(1, 1)
    for cb in range(D_NCH):
        b = cb & 1
        g[b].wait()
        s[b] = store(cb, b)
        if cb + 2 < D_NCH:
            s[b].wait()
            g[b] = gather(cb + 2, b)
    s[0].wait()
    s[1].wait()


def _dispatch_call(xz, dstv, srcv):
    mesh = plsc.VectorSubcoreMesh(core_axis_name="c", subcore_axis_name="s")
    return pl.kernel(
        _dispatch_body,
        out_type=jax.ShapeDtypeStruct((NSLOT, C), jnp.float32),
        mesh=mesh,
        scratch_types=[
            pltpu.VMEM_SHARED((INV_LEN,), jnp.int32),
            pltpu.VMEM((INV_PER_S,), jnp.int32),
            pltpu.VMEM((A_NCH, A_CHUNK), jnp.int32),
            pltpu.VMEM((A_NCH, A_CHUNK), jnp.int32),
            pltpu.VMEM((SLOTS_PER_W,), jnp.int32),
            pltpu.VMEM((2, D_CHUNK, C), jnp.float32),
            pltpu.SemaphoreType.DMA,
            pltpu.SemaphoreType.DMA,
            pltpu.SemaphoreType.DMA,
            pltpu.SemaphoreType.DMA,
        ],
    )(xz, dstv, srcv)


RBLK = 256
NRB = RPE // RBLK


def _glu_body(in_ref, wf_ref, wg_ref, wp_ref, out_ref):
    a = in_ref[0]
    g = jnp.dot(a, wg_ref[0], preferred_element_type=jnp.float32)
    h = jnp.dot(a, wf_ref[0], preferred_element_type=jnp.float32)
    p = jax.nn.silu(g) * h
    out_ref[0] = jnp.dot(p, wp_ref[0], preferred_element_type=jnp.float32)


def _glu_call(ein, w_fc, w_gate, w_proj, interpret=False):
    return pl.pallas_call(
        _glu_body,
        grid=(E, NRB),
        in_specs=[
            pl.BlockSpec((1, RBLK, C), lambda e, r: (e, r, 0)),
            pl.BlockSpec((1, C, H), lambda e, r: (e, 0, 0)),
            pl.BlockSpec((1, C, H), lambda e, r: (e, 0, 0)),
            pl.BlockSpec((1, H, C), lambda e, r: (e, 0, 0)),
        ],
        out_specs=pl.BlockSpec((1, RBLK, C), lambda e, r: (e, r, 0)),
        out_shape=jax.ShapeDtypeStruct((E, RPE, C), jnp.float32),
        interpret=interpret,
    )(ein, w_fc, w_gate, w_proj)


def _combine_body(eo, gi0, gi1, pr0, pr1, y,
                  gi0_v, gi1_v, p0_v, p1_v, g0_v, g1_v, o_v,
                  gsm0, gsm1, ssm0, ssm1):
    cid = lax.axis_index("c")
    sid = lax.axis_index("s")
    wid = sid * NC + cid
    base = wid * TOK_PER_W
    pltpu.sync_copy(gi0.at[pl.ds(base, TOK_PER_W)], gi0_v)
    pltpu.sync_copy(gi1.at[pl.ds(base, TOK_PER_W)], gi1_v)
    pltpu.sync_copy(pr0.at[pl.ds(base * 16, TOK_PER_W * 16)], p0_v)
    pltpu.sync_copy(pr1.at[pl.ds(base * 16, TOK_PER_W * 16)], p1_v)
    gsem = (gsm0, gsm1)
    ssem = (ssm0, ssm1)

    def gathers(cb, b):
        sl = pl.ds(cb * C_CHUNK, C_CHUNK)
        return (pltpu.async_copy(eo.at[gi0_v.at[sl]], g0_v.at[b], gsem[b]),
                pltpu.async_copy(eo.at[gi1_v.at[sl]], g1_v.at[b], gsem[b]))

    g = [gathers(0, 0), gathers(1, 1)]
    s = [None, None]
    for cb in range(C_NCH):
        b = cb & 1
        g[b][0].wait()
        g[b][1].wait()
        if s[b] is not None:
            s[b].wait()

        def tok_body(j, carry, cb=cb, b=b):
            psl = pl.ds((cb * C_CHUNK + j) * 16, 16)
            sp0 = p0_v[psl]
            sp1 = p1_v[psl]
            for ch in range(C // 16):
                sl = pl.ds(ch * 16, 16)
                o_v[b, j, sl] = sp0 * g0_v[b, j, sl] + sp1 * g1_v[b, j, sl]
            return carry

        lax.fori_loop(0, C_CHUNK, tok_body, 0)
        s[b] = pltpu.async_copy(
            o_v.at[b], y.at[pl.ds(base + cb * C_CHUNK, C_CHUNK)], ssem[b])
        if cb + 2 < C_NCH:
            g[b] = gathers(cb + 2, b)
    s[0].wait()
    s[1].wait()


def _combine_call(eo, gi0, gi1, pr0, pr1):
    mesh = plsc.VectorSubcoreMesh(core_axis_name="c", subcore_axis_name="s")
    return pl.kernel(
        _combine_body,
        out_type=jax.ShapeDtypeStruct((NTOK, C), jnp.float32),
        mesh=mesh,
        scratch_types=[
            pltpu.VMEM((TOK_PER_W,), jnp.int32),
            pltpu.VMEM((TOK_PER_W,), jnp.int32),
            pltpu.VMEM((TOK_PER_W * 16,), jnp.float32),
            pltpu.VMEM((TOK_PER_W * 16,), jnp.float32),
            pltpu.VMEM((2, C_CHUNK, C), jnp.float32),
            pltpu.VMEM((2, C_CHUNK, C), jnp.float32),
            pltpu.VMEM((2, C_CHUNK, C), jnp.float32),
            pltpu.SemaphoreType.DMA,
            pltpu.SemaphoreType.DMA,
            pltpu.SemaphoreType.DMA,
            pltpu.SemaphoreType.DMA,
        ],
    )(eo, gi0, gi1, pr0, pr1)


def kernel(x, W_router, w_fc, w_gate, w_proj):
    wp = jnp.pad(W_router, ((0, 0), (0, LANES - E)))
    tri = jnp.tril(jnp.ones((T, T), jnp.float32), -1)
    p0, p1, dst0, dst1, g0, g1 = _router_call(x, wp, tri)
    xz = jnp.concatenate(
        [x.reshape(NTOK, C), jnp.zeros((8, C), x.dtype)], axis=0)
    dstv = jnp.concatenate(
        [dst0.reshape(-1), dst1.reshape(-1)]).reshape(NS * A_NCH, A_CHUNK)
    srcv = jnp.tile(jnp.arange(NTOK, dtype=jnp.int32),
                    TOPK).reshape(NS * A_NCH, A_CHUNK)
    ein = _dispatch_call(xz, dstv, srcv)
    eo = _glu_call(ein.reshape(E, RPE, C), w_fc, w_gate, w_proj)
    p0x = jnp.broadcast_to(p0.reshape(NTOK, 1), (NTOK, 16)).reshape(NTOK * 16)
    p1x = jnp.broadcast_to(p1.reshape(NTOK, 1), (NTOK, 16)).reshape(NTOK * 16)
    y = _combine_call(eo.reshape(NSLOT, C), g0.reshape(-1), g1.reshape(-1),
                      p0x, p1x)
    return y.reshape(B, T, C)


# speedup vs baseline: 1.6362x; 1.4234x over previous
import functools

import jax
import jax.numpy as jnp
from jax import lax
from jax.experimental import pallas as pl
from jax.experimental.pallas import tpu as pltpu
from jax.experimental.pallas import tpu_sc as plsc

B, T, C, E, H = 2, 2048, 1024, 8, 2048
TOPK = 2
CAP = int(1.25 * TOPK * max(1, T / E))
RPE = B * CAP
NSLOT = E * RPE
NTOK = B * T
LANES = 128

NC, NS = 2, 16
NW = NC * NS
RBLK = 256
NSLOT_PAD = NSLOT + RBLK
TRASH = NSLOT
TOK_PER_W = NTOK // NW
D_CHUNK = 32
D_NCH = TOK_PER_W // D_CHUNK
C_CHUNK = 16
C_NCH = TOK_PER_W // C_CHUNK


def _router_body(x_ref, wp_ref, tri_ref,
                 p0_ref, p1_ref, dst0_ref, dst1_ref, g0_ref, g1_ref):
    b = pl.program_id(0)
    xb = x_ref[0]
    logits = jnp.dot(xb, wp_ref[...], preferred_element_type=jnp.float32)
    col = lax.broadcasted_iota(jnp.int32, (T, LANES), 1)
    valid = col < E
    lg = jnp.where(valid, logits, jnp.float32(-1e30))
    m = jnp.max(lg, axis=1, keepdims=True)
    ex = jnp.where(valid, jnp.exp(lg - m), 0.0)
    s = jnp.sum(ex, axis=1, keepdims=True)
    prob = ex / s
    pn = jnp.where(valid, prob, -1.0)
    v1 = jnp.max(pn, axis=1, keepdims=True)
    e0 = jnp.min(jnp.where(valid & (pn == v1), col, 999), axis=1,
                 keepdims=True)
    oh0 = col == e0
    pn2 = jnp.where(oh0, -1.0, pn)
    v2 = jnp.max(pn2, axis=1, keepdims=True)
    e1 = jnp.min(jnp.where(valid & (pn2 == v2), col, 999), axis=1,
                 keepdims=True)
    oh1 = col == e1
    oh0f = oh0.astype(jnp.float32)
    oh1f = oh1.astype(jnp.float32)
    c0 = jnp.dot(tri_ref[...], oh0f, preferred_element_type=jnp.float32)
    c1 = jnp.dot(tri_ref[...], oh1f, preferred_element_type=jnp.float32)
    tot0 = jnp.sum(oh0f, axis=0, keepdims=True)
    pos0 = jnp.sum(c0 * oh0f, axis=1).astype(jnp.int32)
    pos1 = jnp.sum((c1 + tot0) * oh1f, axis=1).astype(jnp.int32)
    e0s = jnp.sum(jnp.where(oh0, col, 0), axis=1)
    e1s = jnp.sum(jnp.where(oh1, col, 0), axis=1)
    base0 = e0s * RPE + b * CAP
    base1 = e1s * RPE + b * CAP
    p0_ref[0, 0, :] = jnp.sum(jnp.where(oh0, prob, 0.0), axis=1)
    p1_ref[0, 0, :] = jnp.sum(jnp.where(oh1, prob, 0.0), axis=1)
    dst0_ref[0, 0, :] = jnp.where(pos0 < CAP, base0 + pos0, TRASH)
    dst1_ref[0, 0, :] = jnp.where(pos1 < CAP, base1 + pos1, TRASH)
    g0_ref[0, 0, :] = base0 + jnp.minimum(pos0, CAP - 1)
    g1_ref[0, 0, :] = base1 + jnp.minimum(pos1, CAP - 1)


def _router_call(x, wp, tri, interpret=False):
    i32 = jnp.int32
    out_shape = [
        jax.ShapeDtypeStruct((B, 1, T), jnp.float32),
        jax.ShapeDtypeStruct((B, 1, T), jnp.float32),
        jax.ShapeDtypeStruct((B, 1, T), i32),
        jax.ShapeDtypeStruct((B, 1, T), i32),
        jax.ShapeDtypeStruct((B, 1, T), i32),
        jax.ShapeDtypeStruct((B, 1, T), i32),
    ]
    ospec = pl.BlockSpec((1, 1, T), lambda b: (b, 0, 0))
    return pl.pallas_call(
        _router_body,
        grid=(B,),
        in_specs=[
            pl.BlockSpec((1, T, C), lambda b: (b, 0, 0)),
            pl.BlockSpec((C, LANES), lambda b: (0, 0)),
            pl.BlockSpec((T, T), lambda b: (0, 0)),
        ],
        out_specs=[ospec] * 6,
        out_shape=out_shape,
        interpret=interpret,
    )(x, wp, tri)


def _dispatch_body(x2d, dst0, dst1, out,
                   d0_v, d1_v, rows_v, gs0, gs1, ss0, ss1):
    cid = lax.axis_index("c")
    sid = lax.axis_index("s")
    wid = sid * NC + cid
    base = wid * TOK_PER_W
    pltpu.sync_copy(dst0.at[pl.ds(base, TOK_PER_W)], d0_v)
    pltpu.sync_copy(dst1.at[pl.ds(base, TOK_PER_W)], d1_v)
    gsem = (gs0, gs1)
    ssem = (ss0, ss1)

    def load(cb, b):
        return pltpu.async_copy(
            x2d.at[pl.ds(base + cb * D_CHUNK, D_CHUNK)], rows_v.at[b],
            gsem[b])

    def scatters(cb, b):
        sl = pl.ds(cb * D_CHUNK, D_CHUNK)
        return (pltpu.async_copy(rows_v.at[b], out.at[d0_v.at[sl]], ssem[b]),
                pltpu.async_copy(rows_v.at[b], out.at[d1_v.at[sl]], ssem[b]))

    g = [load(0, 0), load(1, 1)]
    s = [None, None]
    for cb in range(D_NCH):
        b = cb & 1
        g[b].wait()
        s[b] = scatters(cb, b)
        if cb + 2 < D_NCH:
            s[b][0].wait()
            s[b][1].wait()
            g[b] = load(cb + 2, b)
    for pair in s:
        pair[0].wait()
        pair[1].wait()


def _dispatch_call(x2d, dst0, dst1):
    mesh = plsc.VectorSubcoreMesh(core_axis_name="c", subcore_axis_name="s")
    return pl.kernel(
        _dispatch_body,
        out_type=jax.ShapeDtypeStruct((NSLOT_PAD, C), jnp.float32),
        mesh=mesh,
        scratch_types=[
            pltpu.VMEM((TOK_PER_W,), jnp.int32),
            pltpu.VMEM((TOK_PER_W,), jnp.int32),
            pltpu.VMEM((2, D_CHUNK, C), jnp.float32),
            pltpu.SemaphoreType.DMA,
            pltpu.SemaphoreType.DMA,
            pltpu.SemaphoreType.DMA,
            pltpu.SemaphoreType.DMA,
        ],
    )(x2d, dst0, dst1)


NRB = RPE // RBLK


def _glu_body(in_ref, wf_ref, wg_ref, wp_ref, out_ref):
    a = in_ref[...]
    g = jnp.dot(a, wg_ref[0], preferred_element_type=jnp.float32)
    h = jnp.dot(a, wf_ref[0], preferred_element_type=jnp.float32)
    p = jax.nn.silu(g) * h
    out_ref[...] = jnp.dot(p, wp_ref[0], preferred_element_type=jnp.float32)


def _glu_call(ein, w_fc, w_gate, w_proj, interpret=False):
    return pl.pallas_call(
        _glu_body,
        grid=(E, NRB),
        in_specs=[
            pl.BlockSpec((RBLK, C), lambda e, r: (e * NRB + r, 0)),
            pl.BlockSpec((1, C, H), lambda e, r: (e, 0, 0)),
            pl.BlockSpec((1, C, H), lambda e, r: (e, 0, 0)),
            pl.BlockSpec((1, H, C), lambda e, r: (e, 0, 0)),
        ],
        out_specs=pl.BlockSpec((RBLK, C), lambda e, r: (e * NRB + r, 0)),
        out_shape=jax.ShapeDtypeStruct((NSLOT, C), jnp.float32),
        interpret=interpret,
    )(ein, w_fc, w_gate, w_proj)


def _combine_body(eo, gi0, gi1, pr0, pr1, y,
                  gi0_v, gi1_v, p0_v, p1_v, g0_v, g1_v, o_v,
                  gsm0, gsm1, ssm0, ssm1):
    cid = lax.axis_index("c")
    sid = lax.axis_index("s")
    wid = sid * NC + cid
    base = wid * TOK_PER_W
    pltpu.sync_copy(gi0.at[pl.ds(base, TOK_PER_W)], gi0_v)
    pltpu.sync_copy(gi1.at[pl.ds(base, TOK_PER_W)], gi1_v)
    pltpu.sync_copy(pr0.at[pl.ds(base * 16, TOK_PER_W * 16)], p0_v)
    pltpu.sync_copy(pr1.at[pl.ds(base * 16, TOK_PER_W * 16)], p1_v)
    gsem = (gsm0, gsm1)
    ssem = (ssm0, ssm1)

    def gathers(cb, b):
        sl = pl.ds(cb * C_CHUNK, C_CHUNK)
        return (pltpu.async_copy(eo.at[gi0_v.at[sl]], g0_v.at[b], gsem[b]),
                pltpu.async_copy(eo.at[gi1_v.at[sl]], g1_v.at[b], gsem[b]))

    g = [gathers(0, 0), gathers(1, 1)]
    s = [None, None]
    for cb in range(C_NCH):
        b = cb & 1
        g[b][0].wait()
        g[b][1].wait()
        if s[b] is not None:
            s[b].wait()

        def tok_body(j, carry, cb=cb, b=b):
            psl = pl.ds((cb * C_CHUNK + j) * 16, 16)
            sp0 = p0_v[psl]
            sp1 = p1_v[psl]
            for ch in range(C // 16):
                sl = pl.ds(ch * 16, 16)
                o_v[b, j, sl] = sp0 * g0_v[b, j, sl] + sp1 * g1_v[b, j, sl]
            return carry

        lax.fori_loop(0, C_CHUNK, tok_body, 0)
        s[b] = pltpu.async_copy(
            o_v.at[b], y.at[pl.ds(base + cb * C_CHUNK, C_CHUNK)], ssem[b])
        if cb + 2 < C_NCH:
            g[b] = gathers(cb + 2, b)
    s[0].wait()
    s[1].wait()


def _combine_call(eo, gi0, gi1, pr0, pr1):
    mesh = plsc.VectorSubcoreMesh(core_axis_name="c", subcore_axis_name="s")
    return pl.kernel(
        _combine_body,
        out_type=jax.ShapeDtypeStruct((NTOK, C), jnp.float32),
        mesh=mesh,
        scratch_types=[
            pltpu.VMEM((TOK_PER_W,), jnp.int32),
            pltpu.VMEM((TOK_PER_W,), jnp.int32),
            pltpu.VMEM((TOK_PER_W * 16,), jnp.float32),
            pltpu.VMEM((TOK_PER_W * 16,), jnp.float32),
            pltpu.VMEM((2, C_CHUNK, C), jnp.float32),
            pltpu.VMEM((2, C_CHUNK, C), jnp.float32),
            pltpu.VMEM((2, C_CHUNK, C), jnp.float32),
            pltpu.SemaphoreType.DMA,
            pltpu.SemaphoreType.DMA,
            pltpu.SemaphoreType.DMA,
            pltpu.SemaphoreType.DMA,
        ],
    )(eo, gi0, gi1, pr0, pr1)


def kernel(x, W_router, w_fc, w_gate, w_proj):
    wp = jnp.pad(W_router, ((0, 0), (0, LANES - E)))
    tri = jnp.tril(jnp.ones((T, T), jnp.float32), -1)
    p0, p1, dst0, dst1, g0, g1 = _router_call(x, wp, tri)
    ein = _dispatch_call(x.reshape(NTOK, C), dst0.reshape(-1),
                         dst1.reshape(-1))
    eo = _glu_call(ein, w_fc, w_gate, w_proj)
    p0x = jnp.broadcast_to(p0.reshape(NTOK, 1), (NTOK, 16)).reshape(NTOK * 16)
    p1x = jnp.broadcast_to(p1.reshape(NTOK, 1), (NTOK, 16)).reshape(NTOK * 16)
    y = _combine_call(eo, g0.reshape(-1), g1.reshape(-1), p0x, p1x)
    return y.reshape(B, T, C)


# speedup vs baseline: 1.6386x; 1.0015x over previous
import functools

import jax
import jax.numpy as jnp
from jax import lax
from jax.experimental import pallas as pl
from jax.experimental.pallas import tpu as pltpu
from jax.experimental.pallas import tpu_sc as plsc

B, T, C, E, H = 2, 2048, 1024, 8, 2048
TOPK = 2
CAP = int(1.25 * TOPK * max(1, T / E))
RPE = B * CAP
NSLOT = E * RPE
NTOK = B * T
LANES = 128

NC, NS = 2, 16
NW = NC * NS
RBLK = 256
NSLOT_PAD = NSLOT + RBLK
TRASH = NSLOT
TOK_PER_W = NTOK // NW
D_CHUNK = 32
D_NCH = TOK_PER_W // D_CHUNK
C_CHUNK = 16
C_NCH = TOK_PER_W // C_CHUNK


def _router_body(x_ref, wp_ref, tri_ref,
                 p0_ref, p1_ref, dst0_ref, dst1_ref, g0_ref, g1_ref):
    b = pl.program_id(0)
    xb = x_ref[0]
    logits = jnp.dot(xb, wp_ref[...], preferred_element_type=jnp.float32)
    col = lax.broadcasted_iota(jnp.int32, (T, LANES), 1)
    valid = col < E
    lg = jnp.where(valid, logits, jnp.float32(-1e30))
    m = jnp.max(lg, axis=1, keepdims=True)
    ex = jnp.where(valid, jnp.exp(lg - m), 0.0)
    s = jnp.sum(ex, axis=1, keepdims=True)
    prob = ex / s
    pn = jnp.where(valid, prob, -1.0)
    v1 = jnp.max(pn, axis=1, keepdims=True)
    e0 = jnp.min(jnp.where(valid & (pn == v1), col, 999), axis=1,
                 keepdims=True)
    oh0 = col == e0
    pn2 = jnp.where(oh0, -1.0, pn)
    v2 = jnp.max(pn2, axis=1, keepdims=True)
    e1 = jnp.min(jnp.where(valid & (pn2 == v2), col, 999), axis=1,
                 keepdims=True)
    oh1 = col == e1
    oh0f = oh0.astype(jnp.float32)
    oh1f = oh1.astype(jnp.float32)
    c0 = jnp.dot(tri_ref[...], oh0f, preferred_element_type=jnp.float32)
    c1 = jnp.dot(tri_ref[...], oh1f, preferred_element_type=jnp.float32)
    tot0 = jnp.sum(oh0f, axis=0, keepdims=True)
    pos0 = jnp.sum(c0 * oh0f, axis=1).astype(jnp.int32)
    pos1 = jnp.sum((c1 + tot0) * oh1f, axis=1).astype(jnp.int32)
    e0s = jnp.sum(jnp.where(oh0, col, 0), axis=1)
    e1s = jnp.sum(jnp.where(oh1, col, 0), axis=1)
    base0 = e0s * RPE + b * CAP
    base1 = e1s * RPE + b * CAP
    p0_ref[0, 0, :] = jnp.sum(jnp.where(oh0, prob, 0.0), axis=1)
    p1_ref[0, 0, :] = jnp.sum(jnp.where(oh1, prob, 0.0), axis=1)
    dst0_ref[0, 0, :] = jnp.where(pos0 < CAP, base0 + pos0, TRASH)
    dst1_ref[0, 0, :] = jnp.where(pos1 < CAP, base1 + pos1, TRASH)
    g0_ref[0, 0, :] = base0 + jnp.minimum(pos0, CAP - 1)
    g1_ref[0, 0, :] = base1 + jnp.minimum(pos1, CAP - 1)


def _router_call(x, wp, tri, interpret=False):
    i32 = jnp.int32
    out_shape = [
        jax.ShapeDtypeStruct((B, 1, T), jnp.float32),
        jax.ShapeDtypeStruct((B, 1, T), jnp.float32),
        jax.ShapeDtypeStruct((B, 1, T), i32),
        jax.ShapeDtypeStruct((B, 1, T), i32),
        jax.ShapeDtypeStruct((B, 1, T), i32),
        jax.ShapeDtypeStruct((B, 1, T), i32),
    ]
    ospec = pl.BlockSpec((1, 1, T), lambda b: (b, 0, 0))
    return pl.pallas_call(
        _router_body,
        grid=(B,),
        in_specs=[
            pl.BlockSpec((1, T, C), lambda b: (b, 0, 0)),
            pl.BlockSpec((C, LANES), lambda b: (0, 0)),
            pl.BlockSpec((T, T), lambda b: (0, 0)),
        ],
        out_specs=[ospec] * 6,
        out_shape=out_shape,
        interpret=interpret,
    )(x, wp, tri)


def _dispatch_body(x2d, dst0, dst1, out,
                   d0_v, d1_v, rows_v, gs0, gs1, ss0, ss1):
    cid = lax.axis_index("c")
    sid = lax.axis_index("s")
    wid = sid * NC + cid
    base = wid * TOK_PER_W
    pltpu.sync_copy(dst0.at[pl.ds(base, TOK_PER_W)], d0_v)
    pltpu.sync_copy(dst1.at[pl.ds(base, TOK_PER_W)], d1_v)
    gsem = (gs0, gs1)
    ssem = (ss0, ss1)

    def load(cb, b):
        return pltpu.async_copy(
            x2d.at[pl.ds(base + cb * D_CHUNK, D_CHUNK)], rows_v.at[b],
            gsem[b])

    def scatters(cb, b):
        sl = pl.ds(cb * D_CHUNK, D_CHUNK)
        return (pltpu.async_copy(rows_v.at[b], out.at[d0_v.at[sl]], ssem[b]),
                pltpu.async_copy(rows_v.at[b], out.at[d1_v.at[sl]], ssem[b]))

    g = [load(0, 0), load(1, 1)]
    s = [None, None]
    for cb in range(D_NCH):
        b = cb & 1
        g[b].wait()
        s[b] = scatters(cb, b)
        if cb + 2 < D_NCH:
            s[b][0].wait()
            s[b][1].wait()
            g[b] = load(cb + 2, b)
    for pair in s:
        pair[0].wait()
        pair[1].wait()


def _dispatch_call(x2d, dst0, dst1):
    mesh = plsc.VectorSubcoreMesh(core_axis_name="c", subcore_axis_name="s")
    return pl.kernel(
        _dispatch_body,
        out_type=jax.ShapeDtypeStruct((NSLOT_PAD, C), jnp.float32),
        mesh=mesh,
        scratch_types=[
            pltpu.VMEM((TOK_PER_W,), jnp.int32),
            pltpu.VMEM((TOK_PER_W,), jnp.int32),
            pltpu.VMEM((2, D_CHUNK, C), jnp.float32),
            pltpu.SemaphoreType.DMA,
            pltpu.SemaphoreType.DMA,
            pltpu.SemaphoreType.DMA,
            pltpu.SemaphoreType.DMA,
        ],
    )(x2d, dst0, dst1)


NRB = RPE // RBLK


def _glu_body(in_ref, wf_ref, wg_ref, wp_ref, out_ref):
    bf = jnp.bfloat16
    a = in_ref[...].astype(bf)
    g = jnp.dot(a, wg_ref[0].astype(bf), preferred_element_type=jnp.float32)
    h = jnp.dot(a, wf_ref[0].astype(bf), preferred_element_type=jnp.float32)
    p = (jax.nn.silu(g) * h).astype(bf)
    out_ref[...] = jnp.dot(p, wp_ref[0].astype(bf),
                           preferred_element_type=jnp.float32)


def _glu_call(ein, w_fc, w_gate, w_proj, interpret=False):
    return pl.pallas_call(
        _glu_body,
        grid=(E, NRB),
        in_specs=[
            pl.BlockSpec((RBLK, C), lambda e, r: (e * NRB + r, 0)),
            pl.BlockSpec((1, C, H), lambda e, r: (e, 0, 0)),
            pl.BlockSpec((1, C, H), lambda e, r: (e, 0, 0)),
            pl.BlockSpec((1, H, C), lambda e, r: (e, 0, 0)),
        ],
        out_specs=pl.BlockSpec((RBLK, C), lambda e, r: (e * NRB + r, 0)),
        out_shape=jax.ShapeDtypeStruct((NSLOT, C), jnp.float32),
        interpret=interpret,
    )(ein, w_fc, w_gate, w_proj)


def _combine_body(eo, gi0, gi1, pr0, pr1, y,
                  gi0_v, gi1_v, p0_v, p1_v, g0_v, g1_v, o_v,
                  gsm0, gsm1, ssm0, ssm1):
    cid = lax.axis_index("c")
    sid = lax.axis_index("s")
    wid = sid * NC + cid
    base = wid * TOK_PER_W
    pltpu.sync_copy(gi0.at[pl.ds(base, TOK_PER_W)], gi0_v)
    pltpu.sync_copy(gi1.at[pl.ds(base, TOK_PER_W)], gi1_v)
    pltpu.sync_copy(pr0.at[pl.ds(base * 16, TOK_PER_W * 16)], p0_v)
    pltpu.sync_copy(pr1.at[pl.ds(base * 16, TOK_PER_W * 16)], p1_v)
    gsem = (gsm0, gsm1)
    ssem = (ssm0, ssm1)

    def gathers(cb, b):
        sl = pl.ds(cb * C_CHUNK, C_CHUNK)
        return (pltpu.async_copy(eo.at[gi0_v.at[sl]], g0_v.at[b], gsem[b]),
                pltpu.async_copy(eo.at[gi1_v.at[sl]], g1_v.at[b], gsem[b]))

    g = [gathers(0, 0), gathers(1, 1)]
    s = [None, None]
    for cb in range(C_NCH):
        b = cb & 1
        g[b][0].wait()
        g[b][1].wait()
        if s[b] is not None:
            s[b].wait()

        def tok_body(j, carry, cb=cb, b=b):
            psl = pl.ds((cb * C_CHUNK + j) * 16, 16)
            sp0 = p0_v[psl]
            sp1 = p1_v[psl]
            for ch in range(C // 16):
                sl = pl.ds(ch * 16, 16)
                o_v[b, j, sl] = sp0 * g0_v[b, j, sl] + sp1 * g1_v[b, j, sl]
            return carry

        lax.fori_loop(0, C_CHUNK, tok_body, 0)
        s[b] = pltpu.async_copy(
            o_v.at[b], y.at[pl.ds(base + cb * C_CHUNK, C_CHUNK)], ssem[b])
        if cb + 2 < C_NCH:
            g[b] = gathers(cb + 2, b)
    s[0].wait()
    s[1].wait()


def _combine_call(eo, gi0, gi1, pr0, pr1):
    mesh = plsc.VectorSubcoreMesh(core_axis_name="c", subcore_axis_name="s")
    return pl.kernel(
        _combine_body,
        out_type=jax.ShapeDtypeStruct((NTOK, C), jnp.float32),
        mesh=mesh,
        scratch_types=[
            pltpu.VMEM((TOK_PER_W,), jnp.int32),
            pltpu.VMEM((TOK_PER_W,), jnp.int32),
            pltpu.VMEM((TOK_PER_W * 16,), jnp.float32),
            pltpu.VMEM((TOK_PER_W * 16,), jnp.float32),
            pltpu.VMEM((2, C_CHUNK, C), jnp.float32),
            pltpu.VMEM((2, C_CHUNK, C), jnp.float32),
            pltpu.VMEM((2, C_CHUNK, C), jnp.float32),
            pltpu.SemaphoreType.DMA,
            pltpu.SemaphoreType.DMA,
            pltpu.SemaphoreType.DMA,
            pltpu.SemaphoreType.DMA,
        ],
    )(eo, gi0, gi1, pr0, pr1)


def kernel(x, W_router, w_fc, w_gate, w_proj):
    wp = jnp.pad(W_router, ((0, 0), (0, LANES - E)))
    tri = jnp.tril(jnp.ones((T, T), jnp.float32), -1)
    p0, p1, dst0, dst1, g0, g1 = _router_call(x, wp, tri)
    ein = _dispatch_call(x.reshape(NTOK, C), dst0.reshape(-1),
                         dst1.reshape(-1))
    eo = _glu_call(ein, w_fc, w_gate, w_proj)
    p0x = jnp.broadcast_to(p0.reshape(NTOK, 1), (NTOK, 16)).reshape(NTOK * 16)
    p1x = jnp.broadcast_to(p1.reshape(NTOK, 1), (NTOK, 16)).reshape(NTOK * 16)
    y = _combine_call(eo, g0.reshape(-1), g1.reshape(-1), p0x, p1x)
    return y.reshape(B, T, C)


# speedup vs baseline: 1.8104x; 1.1048x over previous
import functools

import jax
import jax.numpy as jnp
from jax import lax
from jax.experimental import pallas as pl
from jax.experimental.pallas import tpu as pltpu
from jax.experimental.pallas import tpu_sc as plsc

B, T, C, E, H = 2, 2048, 1024, 8, 2048
TOPK = 2
CAP = int(1.25 * TOPK * max(1, T / E))
RPE = B * CAP
NSLOT = E * RPE
NTOK = B * T
LANES = 128

NC, NS = 2, 16
NW = NC * NS
RBLK = 256
NSLOT_PAD = NSLOT + RBLK
TRASH = NSLOT
TOK_PER_W = NTOK // NW
D_CHUNK = 32
D_NCH = TOK_PER_W // D_CHUNK
C_CHUNK = 16
C_NCH = TOK_PER_W // C_CHUNK


def _router_body(x_ref, wp_ref, tri_ref,
                 p0_ref, p1_ref, dst0_ref, dst1_ref, g0_ref, g1_ref):
    b = pl.program_id(0)
    xb = x_ref[0]
    logits = jnp.dot(xb, wp_ref[...], preferred_element_type=jnp.float32)
    col = lax.broadcasted_iota(jnp.int32, (T, LANES), 1)
    valid = col < E
    lg = jnp.where(valid, logits, jnp.float32(-1e30))
    m = jnp.max(lg, axis=1, keepdims=True)
    ex = jnp.where(valid, jnp.exp(lg - m), 0.0)
    s = jnp.sum(ex, axis=1, keepdims=True)
    prob = ex / s
    pn = jnp.where(valid, prob, -1.0)
    v1 = jnp.max(pn, axis=1, keepdims=True)
    e0 = jnp.min(jnp.where(valid & (pn == v1), col, 999), axis=1,
                 keepdims=True)
    oh0 = col == e0
    pn2 = jnp.where(oh0, -1.0, pn)
    v2 = jnp.max(pn2, axis=1, keepdims=True)
    e1 = jnp.min(jnp.where(valid & (pn2 == v2), col, 999), axis=1,
                 keepdims=True)
    oh1 = col == e1
    oh0f = oh0.astype(jnp.float32)
    oh1f = oh1.astype(jnp.float32)
    c0 = jnp.dot(tri_ref[...], oh0f, preferred_element_type=jnp.float32)
    c1 = jnp.dot(tri_ref[...], oh1f, preferred_element_type=jnp.float32)
    tot0 = jnp.sum(oh0f, axis=0, keepdims=True)
    pos0 = jnp.sum(c0 * oh0f, axis=1).astype(jnp.int32)
    pos1 = jnp.sum((c1 + tot0) * oh1f, axis=1).astype(jnp.int32)
    e0s = jnp.sum(jnp.where(oh0, col, 0), axis=1)
    e1s = jnp.sum(jnp.where(oh1, col, 0), axis=1)
    base0 = e0s * RPE + b * CAP
    base1 = e1s * RPE + b * CAP
    p0_ref[0, 0, :] = jnp.sum(jnp.where(oh0, prob, 0.0), axis=1)
    p1_ref[0, 0, :] = jnp.sum(jnp.where(oh1, prob, 0.0), axis=1)
    dst0_ref[0, 0, :] = jnp.where(pos0 < CAP, base0 + pos0, TRASH)
    dst1_ref[0, 0, :] = jnp.where(pos1 < CAP, base1 + pos1, TRASH)
    g0_ref[0, 0, :] = base0 + jnp.minimum(pos0, CAP - 1)
    g1_ref[0, 0, :] = base1 + jnp.minimum(pos1, CAP - 1)


def _router_call(x, wp, tri, interpret=False):
    i32 = jnp.int32
    out_shape = [
        jax.ShapeDtypeStruct((B, 1, T), jnp.float32),
        jax.ShapeDtypeStruct((B, 1, T), jnp.float32),
        jax.ShapeDtypeStruct((B, 1, T), i32),
        jax.ShapeDtypeStruct((B, 1, T), i32),
        jax.ShapeDtypeStruct((B, 1, T), i32),
        jax.ShapeDtypeStruct((B, 1, T), i32),
    ]
    ospec = pl.BlockSpec((1, 1, T), lambda b: (b, 0, 0))
    return pl.pallas_call(
        _router_body,
        grid=(B,),
        in_specs=[
            pl.BlockSpec((1, T, C), lambda b: (b, 0, 0)),
            pl.BlockSpec((C, LANES), lambda b: (0, 0)),
            pl.BlockSpec((T, T), lambda b: (0, 0)),
        ],
        out_specs=[ospec] * 6,
        out_shape=out_shape,
        interpret=interpret,
    )(x, wp, tri)


def _dispatch_body(x2d, dst0, dst1, out,
                   d0_v, d1_v, rows_v, gs0, gs1, ss0, ss1):
    cid = lax.axis_index("c")
    sid = lax.axis_index("s")
    wid = sid * NC + cid
    base = wid * TOK_PER_W
    pltpu.sync_copy(dst0.at[pl.ds(base, TOK_PER_W)], d0_v)
    pltpu.sync_copy(dst1.at[pl.ds(base, TOK_PER_W)], d1_v)
    gsem = (gs0, gs1)
    ssem = (ss0, ss1)

    def load(cb, b):
        return pltpu.async_copy(
            x2d.at[pl.ds(base + cb * D_CHUNK, D_CHUNK)], rows_v.at[b],
            gsem[b])

    def scatters(cb, b):
        sl = pl.ds(cb * D_CHUNK, D_CHUNK)
        return (pltpu.async_copy(rows_v.at[b], out.at[d0_v.at[sl]], ssem[b]),
                pltpu.async_copy(rows_v.at[b], out.at[d1_v.at[sl]], ssem[b]))

    g = [load(0, 0), load(1, 1)]
    s = [None, None]
    for cb in range(D_NCH):
        b = cb & 1
        g[b].wait()
        s[b] = scatters(cb, b)
        if cb + 2 < D_NCH:
            s[b][0].wait()
            s[b][1].wait()
            g[b] = load(cb + 2, b)
    for pair in s:
        pair[0].wait()
        pair[1].wait()


def _dispatch_call(x2d, dst0, dst1):
    mesh = plsc.VectorSubcoreMesh(core_axis_name="c", subcore_axis_name="s")
    return pl.kernel(
        _dispatch_body,
        out_type=jax.ShapeDtypeStruct((NSLOT_PAD, C), jnp.float32),
        mesh=mesh,
        scratch_types=[
            pltpu.VMEM((TOK_PER_W,), jnp.int32),
            pltpu.VMEM((TOK_PER_W,), jnp.int32),
            pltpu.VMEM((2, D_CHUNK, C), jnp.float32),
            pltpu.SemaphoreType.DMA,
            pltpu.SemaphoreType.DMA,
            pltpu.SemaphoreType.DMA,
            pltpu.SemaphoreType.DMA,
        ],
    )(x2d, dst0, dst1)


KH = 4
HK = H // KH


def _glu_body(in_ref, wf_ref, wg_ref, wp_ref, out_ref):
    kh = pl.program_id(1)
    bf = jnp.bfloat16
    a = in_ref[...].astype(bf)
    g = jnp.dot(a, wg_ref[0].astype(bf), preferred_element_type=jnp.float32)
    h = jnp.dot(a, wf_ref[0].astype(bf), preferred_element_type=jnp.float32)
    p = (jax.nn.silu(g) * h).astype(bf)
    part = jnp.dot(p, wp_ref[0].astype(bf),
                   preferred_element_type=jnp.float32)

    @pl.when(kh == 0)
    def _():
        out_ref[...] = part

    @pl.when(kh > 0)
    def _():
        out_ref[...] += part


def _glu_call(ein, w_fc, w_gate, w_proj, interpret=False):
    return pl.pallas_call(
        _glu_body,
        grid=(E, KH),
        in_specs=[
            pl.BlockSpec((RPE, C), lambda e, k: (e, 0)),
            pl.BlockSpec((1, C, HK), lambda e, k: (e, 0, k)),
            pl.BlockSpec((1, C, HK), lambda e, k: (e, 0, k)),
            pl.BlockSpec((1, HK, C), lambda e, k: (e, k, 0)),
        ],
        out_specs=pl.BlockSpec((RPE, C), lambda e, k: (e, 0)),
        out_shape=jax.ShapeDtypeStruct((NSLOT, C), jnp.float32),
        interpret=interpret,
    )(ein, w_fc, w_gate, w_proj)


def _combine_body(eo, gi0, gi1, pr0, pr1, y,
                  gi0_v, gi1_v, p0_v, p1_v, g0_v, g1_v, o_v,
                  gsm0, gsm1, ssm0, ssm1):
    cid = lax.axis_index("c")
    sid = lax.axis_index("s")
    wid = sid * NC + cid
    base = wid * TOK_PER_W
    pltpu.sync_copy(gi0.at[pl.ds(base, TOK_PER_W)], gi0_v)
    pltpu.sync_copy(gi1.at[pl.ds(base, TOK_PER_W)], gi1_v)
    pltpu.sync_copy(pr0.at[pl.ds(base * 16, TOK_PER_W * 16)], p0_v)
    pltpu.sync_copy(pr1.at[pl.ds(base * 16, TOK_PER_W * 16)], p1_v)
    gsem = (gsm0, gsm1)
    ssem = (ssm0, ssm1)

    def gathers(cb, b):
        sl = pl.ds(cb * C_CHUNK, C_CHUNK)
        return (pltpu.async_copy(eo.at[gi0_v.at[sl]], g0_v.at[b], gsem[b]),
                pltpu.async_copy(eo.at[gi1_v.at[sl]], g1_v.at[b], gsem[b]))

    g = [gathers(0, 0), gathers(1, 1)]
    s = [None, None]
    for cb in range(C_NCH):
        b = cb & 1
        g[b][0].wait()
        g[b][1].wait()
        if s[b] is not None:
            s[b].wait()

        def tok_body(j, carry, cb=cb, b=b):
            psl = pl.ds((cb * C_CHUNK + j) * 16, 16)
            sp0 = p0_v[psl]
            sp1 = p1_v[psl]
            for ch in range(C // 16):
                sl = pl.ds(ch * 16, 16)
                o_v[b, j, sl] = sp0 * g0_v[b, j, sl] + sp1 * g1_v[b, j, sl]
            return carry

        lax.fori_loop(0, C_CHUNK, tok_body, 0)
        s[b] = pltpu.async_copy(
            o_v.at[b], y.at[pl.ds(base + cb * C_CHUNK, C_CHUNK)], ssem[b])
        if cb + 2 < C_NCH:
            g[b] = gathers(cb + 2, b)
    s[0].wait()
    s[1].wait()


def _combine_call(eo, gi0, gi1, pr0, pr1):
    mesh = plsc.VectorSubcoreMesh(core_axis_name="c", subcore_axis_name="s")
    return pl.kernel(
        _combine_body,
        out_type=jax.ShapeDtypeStruct((NTOK, C), jnp.float32),
        mesh=mesh,
        scratch_types=[
            pltpu.VMEM((TOK_PER_W,), jnp.int32),
            pltpu.VMEM((TOK_PER_W,), jnp.int32),
            pltpu.VMEM((TOK_PER_W * 16,), jnp.float32),
            pltpu.VMEM((TOK_PER_W * 16,), jnp.float32),
            pltpu.VMEM((2, C_CHUNK, C), jnp.float32),
            pltpu.VMEM((2, C_CHUNK, C), jnp.float32),
            pltpu.VMEM((2, C_CHUNK, C), jnp.float32),
            pltpu.SemaphoreType.DMA,
            pltpu.SemaphoreType.DMA,
            pltpu.SemaphoreType.DMA,
            pltpu.SemaphoreType.DMA,
        ],
    )(eo, gi0, gi1, pr0, pr1)


def kernel(x, W_router, w_fc, w_gate, w_proj):
    wp = jnp.pad(W_router, ((0, 0), (0, LANES - E)))
    tri = jnp.tril(jnp.ones((T, T), jnp.float32), -1)
    p0, p1, dst0, dst1, g0, g1 = _router_call(x, wp, tri)
    ein = _dispatch_call(x.reshape(NTOK, C), dst0.reshape(-1),
                         dst1.reshape(-1))
    eo = _glu_call(ein, w_fc, w_gate, w_proj)
    p0x = jnp.broadcast_to(p0.reshape(NTOK, 1), (NTOK, 16)).reshape(NTOK * 16)
    p1x = jnp.broadcast_to(p1.reshape(NTOK, 1), (NTOK, 16)).reshape(NTOK * 16)
    y = _combine_call(eo, g0.reshape(-1), g1.reshape(-1), p0x, p1x)
    return y.reshape(B, T, C)
